# Initial kernel scaffold; baseline (speedup 1.0000x reference)
import jax
import jax.numpy as jnp
from jax.experimental import pallas as pl


def kernel(x, adj, W1, a1, W2, a2):
    raise NotImplementedError("write your pallas kernel here")


# speedup vs baseline: 2.3472x; 2.3472x over previous
import functools

import jax
import jax.numpy as jnp
from jax import lax
from jax.experimental import pallas as pl

N = 10000
NFEAT = 128
NHID = 64
NCLASS = 32
ALPHA = 0.2
BM = 200


def _prep_kernel(x_ref, w_ref, asrc_ref, adstT_ref, wh_ref, fsrc_ref, fdstT_ref):
    wh = jnp.dot(x_ref[...], w_ref[...], preferred_element_type=jnp.float32)
    wh_ref[...] = wh
    fsrc_ref[...] = jnp.dot(wh, asrc_ref[...], preferred_element_type=jnp.float32)
    fdstT_ref[...] = lax.dot_general(
        adstT_ref[...], wh, (((1,), (1,)), ((), ())),
        preferred_element_type=jnp.float32)


def _prep(x, w, a, out_dim):
    asrc = a[:out_dim]
    adstT = a[out_dim:].T
    n, _ = x.shape
    return pl.pallas_call(
        _prep_kernel,
        out_shape=(
            jax.ShapeDtypeStruct((n, out_dim), jnp.float32),
            jax.ShapeDtypeStruct((n, 1), jnp.float32),
            jax.ShapeDtypeStruct((1, n), jnp.float32),
        ),
    )(x, w, asrc, adstT)


def _layer_kernel(adj_ref, wh_ref, fsrc_ref, fdstT_ref, out_ref, *, apply_elu):
    e = fsrc_ref[...] + fdstT_ref[...]
    e = jnp.where(e >= 0, e, ALPHA * e)
    e = jnp.where(adj_ref[...] > 0, e, jnp.float32(-9e15))
    m = jnp.max(e, axis=1, keepdims=True)
    p = jnp.exp(e - m)
    denom = jnp.sum(p, axis=1, keepdims=True)
    o = jnp.dot(p, wh_ref[...], preferred_element_type=jnp.float32) / denom
    if apply_elu:
        o = jnp.where(o > 0, o, jnp.exp(o) - 1.0)
    out_ref[...] = o


def _layer(adj, wh, fsrc, fdstT, out_dim, apply_elu):
    n = adj.shape[0]
    return pl.pallas_call(
        functools.partial(_layer_kernel, apply_elu=apply_elu),
        grid=(n // BM,),
        in_specs=[
            pl.BlockSpec((BM, n), lambda i: (i, 0)),
            pl.BlockSpec((n, out_dim), lambda i: (0, 0)),
            pl.BlockSpec((BM, 1), lambda i: (i, 0)),
            pl.BlockSpec((1, n), lambda i: (0, 0)),
        ],
        out_specs=pl.BlockSpec((BM, out_dim), lambda i: (i, 0)),
        out_shape=jax.ShapeDtypeStruct((n, out_dim), jnp.float32),
    )(adj, wh, fsrc, fdstT)


@jax.jit
def kernel(x, adj, W1, a1, W2, a2):
    wh1, fsrc1, fdstT1 = _prep(x, W1, a1, NHID)
    h = _layer(adj, wh1, fsrc1, fdstT1, NHID, apply_elu=True)
    wh2, fsrc2, fdstT2 = _prep(h, W2, a2, NCLASS)
    return _layer(adj, wh2, fsrc2, fdstT2, NCLASS, apply_elu=False)


# speedup vs baseline: 2.7848x; 1.1864x over previous
import functools

import jax
import jax.numpy as jnp
from jax import lax
from jax.experimental import pallas as pl

N = 10000
NFEAT = 128
NHID = 64
NCLASS = 32
ALPHA = 0.2
BM = 200


def _prep_kernel(x_ref, w_ref, asrc_ref, adstT_ref,
                 wh_ref, a_ref, b_ref, fdst_ref, fd2_ref, meanwh_ref):
    wh = jnp.dot(x_ref[...], w_ref[...], preferred_element_type=jnp.float32)
    wh_ref[...] = wh
    fsrc = jnp.dot(wh, asrc_ref[...], preferred_element_type=jnp.float32)
    fdst = lax.dot_general(adstT_ref[...], wh, (((1,), (1,)), ((), ())),
                           preferred_element_type=jnp.float32)
    fdst_ref[...] = fdst
    fd2_ref[...] = ALPHA * fdst
    maxd = jnp.max(fdst)
    s = fsrc + maxd
    m = jnp.maximum(s, ALPHA * s)
    a_ref[...] = fsrc - m
    b_ref[...] = ALPHA * fsrc - m
    meanwh_ref[...] = jnp.mean(wh, axis=0, keepdims=True)


def _prep(x, w, a, out_dim):
    asrc = a[:out_dim]
    adstT = a[out_dim:].T
    n, _ = x.shape
    return pl.pallas_call(
        _prep_kernel,
        out_shape=(
            jax.ShapeDtypeStruct((n, out_dim), jnp.float32),
            jax.ShapeDtypeStruct((n, 1), jnp.float32),
            jax.ShapeDtypeStruct((n, 1), jnp.float32),
            jax.ShapeDtypeStruct((1, n), jnp.float32),
            jax.ShapeDtypeStruct((1, n), jnp.float32),
            jax.ShapeDtypeStruct((1, out_dim), jnp.float32),
        ),
    )(x, w, asrc, adstT)


def _layer_kernel(adj_ref, wh_ref, a_ref, b_ref, fdst_ref, fd2_ref, meanwh_ref,
                  out_ref, *, apply_elu):
    t = jnp.maximum(a_ref[...] + fdst_ref[...], b_ref[...] + fd2_ref[...])
    p = jnp.exp(t)
    p = jnp.where(adj_ref[...] > 0, p, 0.0)
    denom = jnp.sum(p, axis=1, keepdims=True)
    o = jnp.dot(p, wh_ref[...], preferred_element_type=jnp.float32)
    o = jnp.where(denom > 0, o / denom, meanwh_ref[...])
    if apply_elu:
        o = jnp.where(o > 0, o, jnp.exp(o) - 1.0)
    out_ref[...] = o


def _layer(adj, prepped, out_dim, apply_elu):
    wh, a, b, fdst, fd2, meanwh = prepped
    n = adj.shape[0]
    return pl.pallas_call(
        functools.partial(_layer_kernel, apply_elu=apply_elu),
        grid=(n // BM,),
        in_specs=[
            pl.BlockSpec((BM, n), lambda i: (i, 0)),
            pl.BlockSpec((n, out_dim), lambda i: (0, 0)),
            pl.BlockSpec((BM, 1), lambda i: (i, 0)),
            pl.BlockSpec((BM, 1), lambda i: (i, 0)),
            pl.BlockSpec((1, n), lambda i: (0, 0)),
            pl.BlockSpec((1, n), lambda i: (0, 0)),
            pl.BlockSpec((1, out_dim), lambda i: (0, 0)),
        ],
        out_specs=pl.BlockSpec((BM, out_dim), lambda i: (i, 0)),
        out_shape=jax.ShapeDtypeStruct((n, out_dim), jnp.float32),
    )(adj, *prepped)


@jax.jit
def kernel(x, adj, W1, a1, W2, a2):
    prepped1 = _prep(x, W1, a1, NHID)
    h = _layer(adj, prepped1, NHID, apply_elu=True)
    prepped2 = _prep(h, W2, a2, NCLASS)
    return _layer(adj, prepped2, NCLASS, apply_elu=False)
